# Initial kernel scaffold; baseline (speedup 1.0000x reference)
import jax
import jax.numpy as jnp
from jax.experimental import pallas as pl


def kernel(u, edge_index, lambda_max, W_cheb, b_cheb, W_src, b_src, W_dst, b_dst, attn):
    raise NotImplementedError("write your pallas kernel here")


# speedup vs baseline: 3.9328x; 3.9328x over previous
import functools

import jax
import jax.numpy as jnp
from jax import lax
from jax.experimental import pallas as pl
from jax.experimental.pallas import tpu as pltpu
from jax.experimental.pallas import tpu_sc as plsc

N = 10000
E = 320000
D = 128
NP_ = 10240
NC, NS, L = 2, 16, 16
NW = NC * NS
EPW = 10240
EP = EPW * NW
CH = 128
NCH = EPW // CH
RPT = NP_ // NS

_f32 = jnp.float32


def _wid():
    c = lax.axis_index("c")
    s = lax.axis_index("s")
    return c, s, c * NS + s


@functools.cache
def _sc_kernels():
    mesh = plsc.VectorSubcoreMesh(core_axis_name="c", subcore_axis_name="s",
                                  num_cores=NC, num_subcores=NS)

    @functools.partial(
        pl.kernel,
        out_type=jax.ShapeDtypeStruct((NC, NP_), _f32),
        mesh=mesh,
        compiler_params=pltpu.CompilerParams(use_tc_tiling_on_sc=False, needs_layout_passes=False),
        scratch_types=[
            pltpu.VMEM((NCH, CH), jnp.int32),
            pltpu.VMEM((CH,), _f32),
            pltpu.VMEM_SHARED((NP_,), _f32),
        ],
    )
    def _sc_deg(dst_hbm, z1_hbm, deg_out, dst_v, ones_v, acc_sh):
        c, s, w = _wid()
        pltpu.sync_copy(dst_hbm.at[w], dst_v)
        for k in range(CH // L):
            ones_v[pl.ds(k * L, L)] = jnp.ones((L,), _f32)
        pltpu.sync_copy(z1_hbm.at[pl.ds(s * RPT, RPT)],
                        acc_sh.at[pl.ds(s * RPT, RPT)])
        plsc.subcore_barrier()

        def chunk(j, carry):
            pltpu.sync_copy(ones_v, acc_sh.at[dst_v.at[j]], add=True)
            return carry

        lax.fori_loop(0, NCH, chunk, 0)
        plsc.subcore_barrier()
        pltpu.sync_copy(acc_sh.at[pl.ds(s * RPT, RPT)],
                        deg_out.at[c, pl.ds(s * RPT, RPT)])

    DH = D // 2

    @functools.partial(
        pl.kernel,
        out_type=jax.ShapeDtypeStruct((NC, 2, NP_, DH), _f32),
        mesh=mesh,
        compiler_params=pltpu.CompilerParams(use_tc_tiling_on_sc=False, needs_layout_passes=False),
        scratch_types=[
            pltpu.VMEM((NCH, CH), jnp.int32),
            pltpu.VMEM((NCH, CH), jnp.int32),
            pltpu.VMEM((CH, DH), _f32),
            pltpu.VMEM((CH, DH), _f32),
            pltpu.VMEM_SHARED((NP_, DH), _f32),
            pltpu.SemaphoreType.DMA,
            pltpu.SemaphoreType.DMA,
        ],
    )
    def _sc_agg(y_lo, y_hi, src_hbm, dst_hbm, zh_hbm, out_hbm,
                src_v, dst_v, buf0, buf1, acc_sh, g0, g1):
        c, s, w = _wid()
        pltpu.sync_copy(src_hbm.at[w], src_v)
        pltpu.sync_copy(dst_hbm.at[w], dst_v)
        for h in range(2):
            y_hbm = (y_lo, y_hi)[h]
            pltpu.sync_copy(zh_hbm.at[pl.ds(s * RPT, RPT)],
                            acc_sh.at[pl.ds(s * RPT, RPT)])
            plsc.subcore_barrier()
            pltpu.async_copy(y_hbm.at[src_v.at[0]], buf0, g0)
            pltpu.async_copy(y_hbm.at[src_v.at[1]], buf1, g1)

            def pair(jj, carry):
                a = 2 * jj
                b = a + 1
                pltpu.make_async_copy(y_hbm.at[src_v.at[a]], buf0, g0).wait()
                pltpu.sync_copy(buf0, acc_sh.at[dst_v.at[a]], add=True)

                @pl.when(jj < NCH // 2 - 1)
                def _():
                    pltpu.async_copy(y_hbm.at[src_v.at[a + 2]], buf0, g0)

                pltpu.make_async_copy(y_hbm.at[src_v.at[b]], buf1, g1).wait()
                pltpu.sync_copy(buf1, acc_sh.at[dst_v.at[b]], add=True)

                @pl.when(jj < NCH // 2 - 1)
                def _():
                    pltpu.async_copy(y_hbm.at[src_v.at[b + 2]], buf1, g1)

                return carry

            lax.fori_loop(0, NCH // 2, pair, 0)
            plsc.subcore_barrier()
            pltpu.sync_copy(acc_sh.at[pl.ds(s * RPT, RPT)],
                            out_hbm.at[c, h, pl.ds(s * RPT, RPT)])

    @functools.partial(
        pl.kernel,
        out_type=(jax.ShapeDtypeStruct((NW, NCH, CH), _f32),
                  jax.ShapeDtypeStruct((NC, NP_), _f32)),
        mesh=mesh,
        compiler_params=pltpu.CompilerParams(use_tc_tiling_on_sc=False, needs_layout_passes=False),
        scratch_types=[
            pltpu.VMEM((NCH, CH), jnp.int32),
            pltpu.VMEM((NCH, CH), jnp.int32),
            pltpu.VMEM((CH, D), _f32),
            pltpu.VMEM((CH, D), _f32),
            pltpu.VMEM((CH, D), _f32),
            pltpu.VMEM((CH, D), _f32),
            pltpu.VMEM((NCH, CH), _f32),
            pltpu.VMEM((D,), _f32),
            pltpu.VMEM_SHARED((NP_,), _f32),
            pltpu.SemaphoreType.DMA,
            pltpu.SemaphoreType.DMA,
            pltpu.SemaphoreType.DMA,
            pltpu.SemaphoreType.DMA,
        ],
    )
    def _sc_edge_e(fs_hbm, fd_hbm, src_hbm, dst_hbm, attn_hbm, z1_hbm,
                   e_out, se_out,
                   src_v, dst_v, fsb0, fsb1, fdb0, fdb1, e_vm, attn_v, acc_sh,
                   gs0, gs1, gd0, gd1):
        c, s, w = _wid()
        pltpu.sync_copy(src_hbm.at[w], src_v)
        pltpu.sync_copy(dst_hbm.at[w], dst_v)
        pltpu.sync_copy(attn_hbm, attn_v)
        pltpu.sync_copy(z1_hbm.at[pl.ds(s * RPT, RPT)],
                        acc_sh.at[pl.ds(s * RPT, RPT)])
        plsc.subcore_barrier()
        pltpu.async_copy(fs_hbm.at[src_v.at[0]], fsb0, gs0)
        pltpu.async_copy(fd_hbm.at[dst_v.at[0]], fdb0, gd0)
        pltpu.async_copy(fs_hbm.at[src_v.at[1]], fsb1, gs1)
        pltpu.async_copy(fd_hbm.at[dst_v.at[1]], fdb1, gd1)

        lane = lax.iota(jnp.int32, L)

        def compute_chunk(j, fsb, fdb):
            def grp16(g, carry):
                def row(rr, vacc):
                    r = g * L + rr
                    acc = jnp.zeros((L,), _f32)
                    for dg in range(D // L):
                        x = fsb[r, pl.ds(dg * L, L)] + fdb[r, pl.ds(dg * L, L)]
                        t = jnp.where(x > 0.0, x, 0.2 * x)
                        acc = acc + t * attn_v[pl.ds(dg * L, L)]
                    return jnp.where(lane == rr, jnp.sum(acc), vacc)

                vacc = lax.fori_loop(0, L, row, jnp.zeros((L,), _f32))
                e_vm[j, pl.ds(g * L, L)] = vacc
                return carry

            lax.fori_loop(0, CH // L, grp16, 0)

        def pair(jj, carry):
            a = 2 * jj
            b = a + 1
            pltpu.make_async_copy(fs_hbm.at[src_v.at[a]], fsb0, gs0).wait()
            pltpu.make_async_copy(fd_hbm.at[dst_v.at[a]], fdb0, gd0).wait()
            compute_chunk(a, fsb0, fdb0)

            @pl.when(jj < NCH // 2 - 1)
            def _():
                pltpu.async_copy(fs_hbm.at[src_v.at[a + 2]], fsb0, gs0)
                pltpu.async_copy(fd_hbm.at[dst_v.at[a + 2]], fdb0, gd0)

            pltpu.make_async_copy(fs_hbm.at[src_v.at[b]], fsb1, gs1).wait()
            pltpu.make_async_copy(fd_hbm.at[dst_v.at[b]], fdb1, gd1).wait()
            compute_chunk(b, fsb1, fdb1)

            @pl.when(jj < NCH // 2 - 1)
            def _():
                pltpu.async_copy(fs_hbm.at[src_v.at[b + 2]], fsb1, gs1)
                pltpu.async_copy(fd_hbm.at[dst_v.at[b + 2]], fdb1, gd1)

            return carry

        lax.fori_loop(0, NCH // 2, pair, 0)

        def se_chunk(j, carry):
            pltpu.sync_copy(e_vm.at[j], acc_sh.at[dst_v.at[j]], add=True)
            return carry

        lax.fori_loop(0, NCH, se_chunk, 0)
        pltpu.sync_copy(e_vm, e_out.at[w])
        plsc.subcore_barrier()
        pltpu.sync_copy(acc_sh.at[pl.ds(s * RPT, RPT)],
                        se_out.at[c, pl.ds(s * RPT, RPT)])

    @functools.partial(
        pl.kernel,
        out_type=(jax.ShapeDtypeStruct((NW, NCH, CH), _f32),
                  jax.ShapeDtypeStruct((NC, NP_), _f32)),
        mesh=mesh,
        compiler_params=pltpu.CompilerParams(use_tc_tiling_on_sc=False, needs_layout_passes=False),
        scratch_types=[
            pltpu.VMEM((NCH, CH), jnp.int32),
            pltpu.VMEM((NCH, CH), _f32),
            pltpu.VMEM((NCH, CH), _f32),
            pltpu.VMEM((NP_,), _f32),
            pltpu.VMEM((NP_,), _f32),
            pltpu.VMEM_SHARED((NP_,), _f32),
        ],
    )
    def _sc_softmax_num(e_hbm, dst_hbm, se_hbm, degc_hbm, z1_hbm,
                        ee_out, s_out,
                        dst_v, e_vm, ee_vm, b_tab, tmp, acc_sh):
        c, s, w = _wid()
        pltpu.sync_copy(dst_hbm.at[w], dst_v)
        pltpu.sync_copy(e_hbm.at[w], e_vm)
        pltpu.sync_copy(se_hbm.at[0], b_tab)
        pltpu.sync_copy(se_hbm.at[1], tmp)

        def add_grp(k, carry):
            b_tab[pl.ds(k * L, L)] = b_tab[pl.ds(k * L, L)] + tmp[pl.ds(k * L, L)]
            return carry

        lax.fori_loop(0, NP_ // L, add_grp, 0)
        pltpu.sync_copy(degc_hbm, tmp)

        def div_grp(k, carry):
            b_tab[pl.ds(k * L, L)] = b_tab[pl.ds(k * L, L)] / tmp[pl.ds(k * L, L)]
            return carry

        lax.fori_loop(0, NP_ // L, div_grp, 0)
        pltpu.sync_copy(z1_hbm.at[pl.ds(s * RPT, RPT)],
                        acc_sh.at[pl.ds(s * RPT, RPT)])
        plsc.subcore_barrier()

        def chunk(j, carry):
            def grp(k, carry2):
                dv = dst_v[j, pl.ds(k * L, L)]
                bv = plsc.load_gather(b_tab, [dv])
                ee_vm[j, pl.ds(k * L, L)] = jnp.exp(
                    e_vm[j, pl.ds(k * L, L)] - bv)
                return carry2

            lax.fori_loop(0, CH // L, grp, 0)
            pltpu.sync_copy(ee_vm.at[j], acc_sh.at[dst_v.at[j]], add=True)
            return carry

        lax.fori_loop(0, NCH, chunk, 0)
        pltpu.sync_copy(ee_vm, ee_out.at[w])
        plsc.subcore_barrier()
        pltpu.sync_copy(acc_sh.at[pl.ds(s * RPT, RPT)],
                        s_out.at[c, pl.ds(s * RPT, RPT)])

    @functools.partial(
        pl.kernel,
        out_type=jax.ShapeDtypeStruct((NC, 2, NP_, DH), _f32),
        mesh=mesh,
        compiler_params=pltpu.CompilerParams(use_tc_tiling_on_sc=False, needs_layout_passes=False),
        scratch_types=[
            pltpu.VMEM((NCH, CH), jnp.int32),
            pltpu.VMEM((NCH, CH), jnp.int32),
            pltpu.VMEM((NCH, CH), _f32),
            pltpu.VMEM((NCH, CH), _f32),
            pltpu.VMEM((NP_,), _f32),
            pltpu.VMEM((NP_,), _f32),
            pltpu.VMEM((CH, DH), _f32),
            pltpu.VMEM((CH, DH), _f32),
            pltpu.VMEM_SHARED((NP_, DH), _f32),
            pltpu.SemaphoreType.DMA,
            pltpu.SemaphoreType.DMA,
        ],
    )
    def _sc_wagg(fs_lo, fs_hi, src_hbm, dst_hbm, ee_hbm, s_hbm, zh_hbm,
                 out_hbm,
                 src_v, dst_v, ee_vm, a_vm, s_tab, tmp, buf0, buf1, acc_sh,
                 g0, g1):
        c, s, w = _wid()
        pltpu.sync_copy(src_hbm.at[w], src_v)
        pltpu.sync_copy(dst_hbm.at[w], dst_v)
        pltpu.sync_copy(ee_hbm.at[w], ee_vm)
        pltpu.sync_copy(s_hbm.at[0], s_tab)
        pltpu.sync_copy(s_hbm.at[1], tmp)

        def add_grp(k, carry):
            s_tab[pl.ds(k * L, L)] = s_tab[pl.ds(k * L, L)] + tmp[pl.ds(k * L, L)]
            return carry

        lax.fori_loop(0, NP_ // L, add_grp, 0)

        def aw_chunk(j, carry):
            def grp(k, carry2):
                dv = dst_v[j, pl.ds(k * L, L)]
                sv = plsc.load_gather(s_tab, [dv])
                a_vm[j, pl.ds(k * L, L)] = ee_vm[j, pl.ds(k * L, L)] / sv
                return carry2

            lax.fori_loop(0, CH // L, grp, 0)
            return carry

        lax.fori_loop(0, NCH, aw_chunk, 0)

        def scale_chunk(j, buf):
            def row(r, carry):
                av = plsc.load_gather(
                    a_vm, [jnp.full((L,), j, jnp.int32),
                           jnp.full((L,), r, jnp.int32)])
                for dg in range(DH // L):
                    buf[r, pl.ds(dg * L, L)] = buf[r, pl.ds(dg * L, L)] * av
                return carry

            lax.fori_loop(0, CH, row, 0)

        for h in range(2):
            fs_hbm = (fs_lo, fs_hi)[h]
            pltpu.sync_copy(zh_hbm.at[pl.ds(s * RPT, RPT)],
                            acc_sh.at[pl.ds(s * RPT, RPT)])
            plsc.subcore_barrier()
            pltpu.async_copy(fs_hbm.at[src_v.at[0]], buf0, g0)
            pltpu.async_copy(fs_hbm.at[src_v.at[1]], buf1, g1)

            def pair(jj, carry):
                a = 2 * jj
                b = a + 1
                pltpu.make_async_copy(fs_hbm.at[src_v.at[a]], buf0, g0).wait()
                scale_chunk(a, buf0)
                pltpu.sync_copy(buf0, acc_sh.at[dst_v.at[a]], add=True)

                @pl.when(jj < NCH // 2 - 1)
                def _():
                    pltpu.async_copy(fs_hbm.at[src_v.at[a + 2]], buf0, g0)

                pltpu.make_async_copy(fs_hbm.at[src_v.at[b]], buf1, g1).wait()
                scale_chunk(b, buf1)
                pltpu.sync_copy(buf1, acc_sh.at[dst_v.at[b]], add=True)

                @pl.when(jj < NCH // 2 - 1)
                def _():
                    pltpu.async_copy(fs_hbm.at[src_v.at[b + 2]], buf1, g1)

                return carry

            lax.fori_loop(0, NCH // 2, pair, 0)
            plsc.subcore_barrier()
            pltpu.sync_copy(acc_sh.at[pl.ds(s * RPT, RPT)],
                            out_hbm.at[c, h, pl.ds(s * RPT, RPT)])

    return _sc_deg, _sc_agg, _sc_edge_e, _sc_softmax_num, _sc_wagg


_BR = 512


def _tc1_body(degp_ref, u_ref, y0_ref, norm_ref, degc_ref):
    deg = degp_ref[0] + degp_ref[1]
    degc = jnp.maximum(deg, 1.0)
    norm = lax.rsqrt(degc)
    degc_ref[...] = degc
    norm_ref[...] = norm
    y0_ref[...] = u_ref[...] * norm


def _tc1(deg_parts, u_pad):
    return pl.pallas_call(
        _tc1_body,
        grid=(NP_ // _BR,),
        in_specs=[
            pl.BlockSpec((2, _BR, 1), lambda i: (0, i, 0)),
            pl.BlockSpec((_BR, D), lambda i: (i, 0)),
        ],
        out_specs=[
            pl.BlockSpec((_BR, D), lambda i: (i, 0)),
            pl.BlockSpec((_BR, 1), lambda i: (i, 0)),
            pl.BlockSpec((_BR, 1), lambda i: (i, 0)),
        ],
        out_shape=[
            jax.ShapeDtypeStruct((NP_, D), _f32),
            jax.ShapeDtypeStruct((NP_, 1), _f32),
            jax.ShapeDtypeStruct((NP_, 1), _f32),
        ],
    )(deg_parts, u_pad)


def _tc2_body(h1p_ref, norm_ref, u_ref, lam_ref, x1_ref, y1_ref):
    rn = 2.0 / lam_ref[0, 0]
    h1 = (h1p_ref[0] + h1p_ref[1]) * norm_ref[...]
    x1 = -rn * h1 + u_ref[...] * (rn - 1.0)
    x1_ref[...] = x1
    y1_ref[...] = x1 * norm_ref[...]


def _tc2(h1_parts, norm, u_pad, lam):
    return pl.pallas_call(
        _tc2_body,
        grid=(NP_ // _BR,),
        in_specs=[
            pl.BlockSpec((2, _BR, D), lambda i: (0, i, 0)),
            pl.BlockSpec((_BR, 1), lambda i: (i, 0)),
            pl.BlockSpec((_BR, D), lambda i: (i, 0)),
            pl.BlockSpec((1, 1), lambda i: (0, 0)),
        ],
        out_specs=[
            pl.BlockSpec((_BR, D), lambda i: (i, 0)),
            pl.BlockSpec((_BR, D), lambda i: (i, 0)),
        ],
        out_shape=[
            jax.ShapeDtypeStruct((NP_, D), _f32),
            jax.ShapeDtypeStruct((NP_, D), _f32),
        ],
    )(h1_parts, norm, u_pad, lam)


def _tc3_body(h2p_ref, norm_ref, x1_ref, u_ref, lam_ref,
              w0_ref, w1_ref, w2_ref, bc_ref, ws_ref, bs_ref, wd_ref, bd_ref,
              fs_ref, fd_ref):
    rn = 2.0 / lam_ref[0, 0]
    h2 = (h2p_ref[0] + h2p_ref[1]) * norm_ref[...]
    x1 = x1_ref[...]
    u = u_ref[...]
    x2 = -2.0 * rn * h2 + x1 * (2.0 * rn - 1.0) - u
    h = (jnp.dot(u, w0_ref[...], preferred_element_type=_f32)
         + jnp.dot(x1, w1_ref[...], preferred_element_type=_f32)
         + jnp.dot(x2, w2_ref[...], preferred_element_type=_f32)
         + bc_ref[...])
    h = jnp.maximum(h, 0.0)
    fs_ref[...] = jnp.dot(h, ws_ref[...], preferred_element_type=_f32) + bs_ref[...]
    fd_ref[...] = jnp.dot(h, wd_ref[...], preferred_element_type=_f32) + bd_ref[...]


def _tc3(h2_parts, norm, x1, u_pad, lam, w0, w1, w2, bc, ws, bs, wd, bd):
    full = lambda i: (0, 0)
    return pl.pallas_call(
        _tc3_body,
        grid=(NP_ // _BR,),
        in_specs=[
            pl.BlockSpec((2, _BR, D), lambda i: (0, i, 0)),
            pl.BlockSpec((_BR, 1), lambda i: (i, 0)),
            pl.BlockSpec((_BR, D), lambda i: (i, 0)),
            pl.BlockSpec((_BR, D), lambda i: (i, 0)),
            pl.BlockSpec((1, 1), full),
            pl.BlockSpec((D, D), full),
            pl.BlockSpec((D, D), full),
            pl.BlockSpec((D, D), full),
            pl.BlockSpec((1, D), full),
            pl.BlockSpec((D, D), full),
            pl.BlockSpec((1, D), full),
            pl.BlockSpec((D, D), full),
            pl.BlockSpec((1, D), full),
        ],
        out_specs=[
            pl.BlockSpec((_BR, D), lambda i: (i, 0)),
            pl.BlockSpec((_BR, D), lambda i: (i, 0)),
        ],
        out_shape=[
            jax.ShapeDtypeStruct((NP_, D), _f32),
            jax.ShapeDtypeStruct((NP_, D), _f32),
        ],
    )(h2_parts, norm, x1, u_pad, lam, w0, w1, w2, bc, ws, bs, wd, bd)


def _tc4_body(op_ref, out_ref):
    out_ref[...] = op_ref[0] + op_ref[1]


def _tc4(out_parts):
    return pl.pallas_call(
        _tc4_body,
        grid=(NP_ // _BR,),
        in_specs=[pl.BlockSpec((2, _BR, D), lambda i: (0, i, 0))],
        out_specs=pl.BlockSpec((_BR, D), lambda i: (i, 0)),
        out_shape=jax.ShapeDtypeStruct((NP_, D), _f32),
    )(out_parts)


def kernel(u, edge_index, lambda_max, W_cheb, b_cheb, W_src, b_src,
           W_dst, b_dst, attn):
    sc_deg, sc_agg, sc_edge_e, sc_softmax_num, sc_wagg = _sc_kernels()

    u_pad = jnp.pad(u, ((0, NP_ - N), (0, 0)))
    pad_e = EP - E
    src = jnp.concatenate([edge_index[0],
                           jnp.full((pad_e,), NP_ - 1, jnp.int32)])
    dst = jnp.concatenate([edge_index[1],
                           jnp.full((pad_e,), NP_ - 1, jnp.int32)])
    src2d = src.reshape(NW, NCH, CH)
    dst2d = dst.reshape(NW, NCH, CH)
    z1 = jnp.zeros((NP_,), _f32)
    zh = jnp.zeros((NP_, D // 2), _f32)
    lam = lambda_max.reshape(1, 1)
    w0 = W_cheb[0 * D:1 * D]
    w1 = W_cheb[1 * D:2 * D]
    w2 = W_cheb[2 * D:3 * D]
    bc = b_cheb.reshape(1, D)
    bs = b_src.reshape(1, D)
    bd = b_dst.reshape(1, D)
    attn_v = attn.reshape(D)

    def _halves(x):
        return x[:, :D // 2], x[:, D // 2:]

    def _merge(parts):
        return jnp.concatenate([parts[:, 0], parts[:, 1]], axis=-1)

    deg_parts = sc_deg(dst2d, z1)
    y0, norm, degc = _tc1(deg_parts.reshape(2, NP_, 1), u_pad)
    h1_parts = sc_agg(*_halves(y0), src2d, dst2d, zh)
    x1, y1 = _tc2(_merge(h1_parts), norm, u_pad, lam)
    h2_parts = sc_agg(*_halves(y1), src2d, dst2d, zh)
    fs, fd = _tc3(_merge(h2_parts), norm, x1, u_pad, lam, w0, w1, w2, bc,
                  W_src, bs, W_dst, bd)

    e_edges, se_parts = sc_edge_e(fs, fd, src2d, dst2d, attn_v, z1)
    ee_edges, s_parts = sc_softmax_num(e_edges, dst2d, se_parts,
                                       degc.reshape(NP_), z1)
    out_parts = sc_wagg(*_halves(fs), src2d, dst2d, ee_edges, s_parts, zh)
    out = _tc4(_merge(out_parts))
    return out[:N]
